# baseline (speedup 1.0000x reference)
import functools

import jax
import jax.numpy as jnp
from jax import lax
from jax.experimental import pallas as pl
from jax.experimental.pallas import tpu as pltpu
from jax.experimental.pallas import tpu_sc as plsc

N = 10000
D = 128
DH = D // 2
E = 320000
EP = 327680
PAD_E = EP - E

NC = 2
NS = 16
NW = NC * NS
RC = 128
NCH1 = EP // NW // RC
NCH3 = EP // NS // RC
NPD = 10240
RPT = NPD // NS
RCH = RPT // RC
NB = 4

_mesh = plsc.VectorSubcoreMesh(
    core_axis_name="c", subcore_axis_name="s", num_cores=NC, num_subcores=NS)


def _deg_body(dst_hbm, zeros_hbm, deg_out, idx_d, ones_v, zbuf, dbuf, deg_sh):
    cid = lax.axis_index("c")
    tid = lax.axis_index("s")
    w = cid * NS + tid

    def fill_ones(i, _):
        ones_v[pl.ds(i * 16, 16)] = jnp.full((16,), 1.0, jnp.float32)
        return 0
    lax.fori_loop(0, RC // 16, fill_ones, 0)

    pltpu.sync_copy(dst_hbm.at[w], idx_d)
    pltpu.sync_copy(zeros_hbm, zbuf)
    pltpu.sync_copy(zbuf, deg_sh.at[pl.ds(tid * RPT, RPT)])
    plsc.subcore_barrier()

    def step(c, _):
        pltpu.sync_copy(ones_v, deg_sh.at[idx_d.at[c]], add=True)
        return 0
    lax.fori_loop(0, NCH1, step, 0)
    plsc.subcore_barrier()

    pltpu.sync_copy(deg_sh.at[pl.ds(tid * RPT, RPT)], dbuf)
    pltpu.sync_copy(dbuf, deg_out.at[cid].at[pl.ds(tid * RPT, RPT)])


_deg_kernel = functools.partial(
    pl.kernel,
    out_type=jax.ShapeDtypeStruct((NC, NPD), jnp.float32),
    mesh=_mesh,
    scratch_types=[
        pltpu.VMEM((NCH1, RC), jnp.int32),
        pltpu.VMEM((RC,), jnp.float32),
        pltpu.VMEM((RPT,), jnp.float32),
        pltpu.VMEM((RPT,), jnp.float32),
        pltpu.VMEM_SHARED((NPD,), jnp.float32),
    ],
)(_deg_body)


def _mm_body(x_ref, w_ref, d0_ref, d1_ref, xw_ref, dinv_ref):
    deg = d0_ref[...] + d1_ref[...] + 1.0
    dinv_ref[...] = lax.rsqrt(deg)
    xw_ref[...] = jnp.dot(x_ref[...], w_ref[...],
                          preferred_element_type=jnp.float32)


_BM = 2048


def _mm(x_pad, W, d0, d1):
    return pl.pallas_call(
        _mm_body,
        grid=(NPD // _BM,),
        in_specs=[
            pl.BlockSpec((_BM, D), lambda i: (i, 0)),
            pl.BlockSpec((D, D), lambda i: (0, 0)),
            pl.BlockSpec((_BM, 1), lambda i: (i, 0)),
            pl.BlockSpec((_BM, 1), lambda i: (i, 0)),
        ],
        out_specs=[
            pl.BlockSpec((_BM, D), lambda i: (i, 0)),
            pl.BlockSpec((_BM, 1), lambda i: (i, 0)),
        ],
        out_shape=[
            jax.ShapeDtypeStruct((NPD, D), jnp.float32),
            jax.ShapeDtypeStruct((NPD, 1), jnp.float32),
        ],
    )(x_pad, W, d0, d1)


def _scat_body(src_hbm, dst_hbm, xw_hbm, dinv_hbm, b_hbm,
               out_hbm, z_hbm,
               idx_s, idx_d, rows_a, rows_b, rows_c, rows_d,
               zbuf, dv, bbuf, acc_sh,
               sem_a, sem_b, sem_c, sem_d):
    cid = lax.axis_index("c")
    tid = lax.axis_index("s")
    c0 = cid * DH

    pltpu.sync_copy(src_hbm.at[tid], idx_s)
    pltpu.sync_copy(dst_hbm.at[tid], idx_d)
    pltpu.sync_copy(dinv_hbm.at[pl.ds(tid * RPT, RPT)], dv)
    pltpu.sync_copy(b_hbm.at[cid], bbuf)
    bv = [bbuf[pl.ds(j * 16, 16)] for j in range(DH // 16)]

    def scale_chunk(r, _):
        base = tid * RPT + r * RC
        pltpu.sync_copy(xw_hbm.at[pl.ds(base, RC), pl.ds(c0, DH)], zbuf)

        def grp(g, _):
            dvv = dv[pl.ds(r * RC + g * 16, 16)]
            for k in range(16):
                sv = lax.broadcast(dvv[k], (16,))
                i = g * 16 + k
                for j in range(DH // 16):
                    zbuf[i, pl.ds(j * 16, 16)] = (
                        zbuf[i, pl.ds(j * 16, 16)] * sv)
            return 0
        lax.fori_loop(0, RC // 16, grp, 0)
        pltpu.sync_copy(zbuf, z_hbm.at[cid].at[pl.ds(base, RC)])
        pltpu.sync_copy(zbuf, acc_sh.at[pl.ds(base, RC)])
        return 0
    lax.fori_loop(0, RCH, scale_chunk, 0)
    plsc.subcore_barrier()

    zsc = z_hbm.at[cid]
    bufs = (rows_a, rows_b, rows_c, rows_d)
    sems = (sem_a, sem_b, sem_c, sem_d)
    for j in range(NB):
        pltpu.async_copy(zsc.at[idx_s.at[j]], bufs[j], sems[j])

    def quad(i, _):
        base = NB * i
        for j in range(NB):
            c = base + j
            pltpu.make_async_copy(zsc.at[idx_s.at[c]], bufs[j], sems[j]).wait()
            pltpu.sync_copy(bufs[j], acc_sh.at[idx_d.at[c]], add=True)

            @pl.when(c + NB < NCH3)
            def _():
                pltpu.async_copy(zsc.at[idx_s.at[c + NB]], bufs[j], sems[j])
        return 0
    lax.fori_loop(0, NCH3 // NB, quad, 0)
    plsc.subcore_barrier()

    def fin_chunk(r, _):
        base = tid * RPT + r * RC
        pltpu.sync_copy(acc_sh.at[pl.ds(base, RC)], zbuf)

        def grp(g, _):
            dvv = dv[pl.ds(r * RC + g * 16, 16)]
            for k in range(16):
                sv = lax.broadcast(dvv[k], (16,))
                i = g * 16 + k
                for j in range(DH // 16):
                    zbuf[i, pl.ds(j * 16, 16)] = (
                        zbuf[i, pl.ds(j * 16, 16)] * sv + bv[j])
            return 0
        lax.fori_loop(0, RC // 16, grp, 0)

        @pl.when(base + RC <= N)
        def _():
            pltpu.sync_copy(zbuf, out_hbm.at[pl.ds(base, RC), pl.ds(c0, DH)])

        @pl.when(base == N - 16)
        def _():
            pltpu.sync_copy(zbuf.at[pl.ds(0, 16)],
                            out_hbm.at[pl.ds(base, 16), pl.ds(c0, DH)])
        return 0
    lax.fori_loop(0, RCH, fin_chunk, 0)


_scat_kernel = functools.partial(
    pl.kernel,
    out_type=[
        jax.ShapeDtypeStruct((N, D), jnp.float32),
        jax.ShapeDtypeStruct((NC, NPD, DH), jnp.float32),
    ],
    mesh=_mesh,
    scratch_types=[
        pltpu.VMEM((NCH3, RC), jnp.int32),
        pltpu.VMEM((NCH3, RC), jnp.int32),
        pltpu.VMEM((RC, DH), jnp.float32),
        pltpu.VMEM((RC, DH), jnp.float32),
        pltpu.VMEM((RC, DH), jnp.float32),
        pltpu.VMEM((RC, DH), jnp.float32),
        pltpu.VMEM((RC, DH), jnp.float32),
        pltpu.VMEM((RPT,), jnp.float32),
        pltpu.VMEM((DH,), jnp.float32),
        pltpu.VMEM_SHARED((NPD, DH), jnp.float32),
        pltpu.SemaphoreType.DMA,
        pltpu.SemaphoreType.DMA,
        pltpu.SemaphoreType.DMA,
        pltpu.SemaphoreType.DMA,
    ],
    compiler_params=pltpu.CompilerParams(use_tc_tiling_on_sc=False),
)(_scat_body)


def kernel(x, edge_index, W, b):
    ei = edge_index.astype(jnp.int32)
    src_p = jnp.concatenate([ei[0], jnp.full((PAD_E,), N, jnp.int32)])
    dst_p = jnp.concatenate([ei[1], jnp.full((PAD_E,), NPD - 1, jnp.int32)])
    dst1 = dst_p.reshape(NW, NCH1, RC)
    src3 = src_p.reshape(NS, NCH3, RC)
    dst3 = dst_p.reshape(NS, NCH3, RC)
    zeros_deg = jnp.zeros((RPT,), jnp.float32)
    x_pad = jnp.pad(x, ((0, NPD - N), (0, 0)))

    deg_p = _deg_kernel(dst1, zeros_deg)
    d0 = deg_p[0].reshape(NPD, 1)
    d1 = deg_p[1].reshape(NPD, 1)

    xw, dinv = _mm(x_pad, W, d0, d1)

    out, _ = _scat_kernel(src3, dst3, xw, dinv.reshape(NPD), b.reshape(NC, DH))
    return out


# speedup vs baseline: 2.2455x; 2.2455x over previous
import functools

import jax
import jax.numpy as jnp
from jax import lax
from jax.experimental import pallas as pl
from jax.experimental.pallas import tpu as pltpu
from jax.experimental.pallas import tpu_sc as plsc

N = 10000
D = 128
DH = D // 2
E = 320000

NC = 2
NS = 16
NW = NC * NS
EPW = E // NW
CHUNK = 125
NCH = EPW // CHUNK
EPT = E // NS
NCH2 = EPT // CHUNK
NPD = 10240
DEG_PT = NPD // NS
RC = 128
RCH = DEG_PT // RC

_mesh = plsc.VectorSubcoreMesh(
    core_axis_name="c", subcore_axis_name="s", num_cores=NC, num_subcores=NS)


def _deg_body(dst_hbm, zeros_hbm, deg_out, idx_d, ones_v, zbuf, dbuf, deg_sh):
    cid = lax.axis_index("c")
    tid = lax.axis_index("s")
    w = cid * NS + tid

    def fill_ones(i, _):
        ones_v[pl.ds(i * 16, 16)] = jnp.full((16,), 1.0, jnp.float32)
        return 0
    lax.fori_loop(0, 8, fill_ones, 0)

    pltpu.sync_copy(dst_hbm.at[w], idx_d)
    pltpu.sync_copy(zeros_hbm, zbuf)
    pltpu.sync_copy(zbuf, deg_sh.at[pl.ds(tid * DEG_PT, DEG_PT)])
    plsc.subcore_barrier()

    def step(c, _):
        pltpu.sync_copy(ones_v.at[pl.ds(0, CHUNK)],
                        deg_sh.at[idx_d.at[c]], add=True)
        return 0
    lax.fori_loop(0, NCH, step, 0)
    plsc.subcore_barrier()

    pltpu.sync_copy(deg_sh.at[pl.ds(tid * DEG_PT, DEG_PT)], dbuf)
    pltpu.sync_copy(dbuf, deg_out.at[cid].at[pl.ds(tid * DEG_PT, DEG_PT)])


_deg_kernel = functools.partial(
    pl.kernel,
    out_type=jax.ShapeDtypeStruct((NC, NPD), jnp.float32),
    mesh=_mesh,
    scratch_types=[
        pltpu.VMEM((NCH, CHUNK), jnp.int32),
        pltpu.VMEM((128,), jnp.float32),
        pltpu.VMEM((DEG_PT,), jnp.float32),
        pltpu.VMEM((DEG_PT,), jnp.float32),
        pltpu.VMEM_SHARED((NPD,), jnp.float32),
    ],
)(_deg_body)


def _mm_body(x_ref, w_ref, d0_ref, d1_ref, z_ref, dinv_ref):
    deg = d0_ref[...] + d1_ref[...] + 1.0
    dinv = lax.rsqrt(deg)
    xw = jnp.dot(x_ref[...], w_ref[...], preferred_element_type=jnp.float32)
    z = xw * dinv
    z_ref[0] = z[:, :DH]
    z_ref[1] = z[:, DH:]
    dinv_ref[...] = dinv


_BM = 2000


def _mm(x, W, d0, d1):
    return pl.pallas_call(
        _mm_body,
        grid=(N // _BM,),
        in_specs=[
            pl.BlockSpec((_BM, D), lambda i: (i, 0)),
            pl.BlockSpec((D, D), lambda i: (0, 0)),
            pl.BlockSpec((_BM, 1), lambda i: (i, 0)),
            pl.BlockSpec((_BM, 1), lambda i: (i, 0)),
        ],
        out_specs=[
            pl.BlockSpec((NC, _BM, DH), lambda i: (0, i, 0)),
            pl.BlockSpec((_BM, 1), lambda i: (i, 0)),
        ],
        out_shape=[
            jax.ShapeDtypeStruct((NC, N, DH), jnp.float32),
            jax.ShapeDtypeStruct((N, 1), jnp.float32),
        ],
    )(x, W, d0, d1)


def _scat_body(src_hbm, dst_hbm, z_hbm, zrows_hbm, acc_out,
               idx_s, idx_d, rows_a, rows_b, rows_c, rows_d, zbuf, acc_sh,
               sem_a, sem_b, sem_c, sem_d):
    cid = lax.axis_index("c")
    tid = lax.axis_index("s")

    pltpu.sync_copy(src_hbm.at[tid], idx_s)
    pltpu.sync_copy(dst_hbm.at[tid], idx_d)
    pltpu.sync_copy(zrows_hbm, zbuf)

    def init_step(r, _):
        pltpu.sync_copy(zbuf, acc_sh.at[pl.ds(tid * DEG_PT + r * RC, RC)])
        return 0
    lax.fori_loop(0, RCH, init_step, 0)
    plsc.subcore_barrier()

    zsc = z_hbm.at[cid]
    bufs = (rows_a, rows_b, rows_c, rows_d)
    sems = (sem_a, sem_b, sem_c, sem_d)
    NB = 4
    for j in range(NB):
        pltpu.async_copy(zsc.at[idx_s.at[j]], bufs[j], sems[j])

    def quad(i, _):
        base = NB * i
        for j in range(NB):
            c = base + j
            pltpu.make_async_copy(zsc.at[idx_s.at[c]], bufs[j], sems[j]).wait()
            pltpu.sync_copy(bufs[j], acc_sh.at[idx_d.at[c]], add=True)

            @pl.when(c + NB < NCH2)
            def _():
                pltpu.async_copy(zsc.at[idx_s.at[c + NB]], bufs[j], sems[j])
        return 0
    lax.fori_loop(0, NCH2 // NB, quad, 0)
    plsc.subcore_barrier()

    def out_step(r, _):
        base = tid * DEG_PT + r * RC
        pltpu.sync_copy(acc_sh.at[pl.ds(base, RC)], zbuf)
        pltpu.sync_copy(zbuf, acc_out.at[cid].at[pl.ds(base, RC)])
        return 0
    lax.fori_loop(0, RCH, out_step, 0)


_scat_kernel = functools.partial(
    pl.kernel,
    out_type=jax.ShapeDtypeStruct((NC, NPD, DH), jnp.float32),
    mesh=_mesh,
    scratch_types=[
        pltpu.VMEM((NCH2, CHUNK), jnp.int32),
        pltpu.VMEM((NCH2, CHUNK), jnp.int32),
        pltpu.VMEM((CHUNK, DH), jnp.float32),
        pltpu.VMEM((CHUNK, DH), jnp.float32),
        pltpu.VMEM((CHUNK, DH), jnp.float32),
        pltpu.VMEM((CHUNK, DH), jnp.float32),
        pltpu.VMEM((RC, DH), jnp.float32),
        pltpu.VMEM_SHARED((NPD, DH), jnp.float32),
        pltpu.SemaphoreType.DMA,
        pltpu.SemaphoreType.DMA,
        pltpu.SemaphoreType.DMA,
        pltpu.SemaphoreType.DMA,
    ],
    compiler_params=pltpu.CompilerParams(use_tc_tiling_on_sc=False),
)(_scat_body)


def _out_body(a_ref, z_ref, dinv_ref, b_ref, o_ref):
    a = jnp.concatenate([a_ref[0], a_ref[1]], axis=1)
    z = jnp.concatenate([z_ref[0], z_ref[1]], axis=1)
    o_ref[...] = (a + z) * dinv_ref[...] + b_ref[...]


def _combine(acc, z, dinv, b2):
    return pl.pallas_call(
        _out_body,
        grid=(N // _BM,),
        in_specs=[
            pl.BlockSpec((NC, _BM, DH), lambda i: (0, i, 0)),
            pl.BlockSpec((NC, _BM, DH), lambda i: (0, i, 0)),
            pl.BlockSpec((_BM, 1), lambda i: (i, 0)),
            pl.BlockSpec((1, D), lambda i: (0, 0)),
        ],
        out_specs=pl.BlockSpec((_BM, D), lambda i: (i, 0)),
        out_shape=jax.ShapeDtypeStruct((N, D), jnp.float32),
    )(acc, z, dinv, b2)


def kernel(x, edge_index, W, b):
    ei = edge_index.astype(jnp.int32)
    src_deg = ei[0].reshape(NW, NCH, CHUNK)
    dst_deg = ei[1].reshape(NW, NCH, CHUNK)
    src_sc = ei[0].reshape(NS, NCH2, CHUNK)
    dst_sc = ei[1].reshape(NS, NCH2, CHUNK)
    zeros_deg = jnp.zeros((DEG_PT,), jnp.float32)
    zeros_rows = jnp.zeros((RC, DH), jnp.float32)

    deg_p = _deg_kernel(dst_deg, zeros_deg)
    d0 = deg_p[0, :N].reshape(N, 1)
    d1 = deg_p[1, :N].reshape(N, 1)

    z, dinv = _mm(x, W, d0, d1)

    acc = _scat_kernel(src_sc, dst_sc, z, zeros_rows)

    out = _combine(acc, z, dinv, b.reshape(1, D))
    return out
